# baseline (speedup 1.0000x reference)
import functools

import jax
import jax.numpy as jnp
from jax import lax
from jax.experimental import pallas as pl
from jax.experimental.pallas import tpu as pltpu
from jax.experimental.pallas import tpu_sc as plsc

N = 10000
E = 160000
D = 256
H = 128
NT = 16
EPT = E // NT
RB = 624
CH = 128
NF = 78
NPAIR = NF // 2
EPT2 = E // (2 * NT)
NF2 = 39
BM = 1000

@functools.cache
def _mesh():
    return plsc.VectorSubcoreMesh(
        core_axis_name="c", subcore_axis_name="s", num_cores=2, num_subcores=NT)


def _mm_body(h_ref, w_ref, dega_ref, degb_ref, y0_ref, y1_ref):
    xw = jnp.dot(h_ref[...], w_ref[...], preferred_element_type=jnp.float32)
    dinv = lax.rsqrt(dega_ref[:, 0:1] + degb_ref[:, 0:1] + 1.0)
    y = xw * dinv
    y0_ref[...] = y[:, :H]
    y1_ref[...] = y[:, H:]


def _matmul_scale(h, w, dega, degb):
    return pl.pallas_call(
        _mm_body,
        grid=(N // BM,),
        in_specs=[
            pl.BlockSpec((BM, D), lambda i: (i, 0)),
            pl.BlockSpec((D, D), lambda i: (0, 0)),
            pl.BlockSpec((BM, H), lambda i: (i, 0)),
            pl.BlockSpec((BM, H), lambda i: (i, 0)),
        ],
        out_specs=[pl.BlockSpec((BM, H), lambda i: (i, 0))] * 2,
        out_shape=[jax.ShapeDtypeStruct((N, H), jnp.float32)] * 2,
    )(h, w, dega, degb)


def _cmm_body(a0_ref, a1_ref, y0_ref, y1_ref, dega_ref, degb_ref, b_ref, w_ref,
              y0o_ref, y1o_ref):
    dinv = lax.rsqrt(dega_ref[:, 0:1] + degb_ref[:, 0:1] + 1.0)
    bv = b_ref[...]
    left = jnp.maximum((a0_ref[...] + y0_ref[...]) * dinv + bv[:, :H], 0.0)
    right = jnp.maximum((a1_ref[...] + y1_ref[...]) * dinv + bv[:, H:], 0.0)
    h = jnp.concatenate([left, right], axis=1)
    y = jnp.dot(h, w_ref[...], preferred_element_type=jnp.float32) * dinv
    y0o_ref[...] = y[:, :H]
    y1o_ref[...] = y[:, H:]


def _combine_matmul(a0, a1, y0, y1, dega, degb, b2d, w):
    blk = pl.BlockSpec((BM, H), lambda i: (i, 0))
    return pl.pallas_call(
        _cmm_body,
        grid=(N // BM,),
        in_specs=[
            blk, blk, blk, blk, blk, blk,
            pl.BlockSpec((1, D), lambda i: (0, 0)),
            pl.BlockSpec((D, D), lambda i: (0, 0)),
        ],
        out_specs=[blk, blk],
        out_shape=[jax.ShapeDtypeStruct((N, H), jnp.float32)] * 2,
    )(a0, a1, y0, y1, dega, degb, b2d, w)


def _comb_body(a0_ref, a1_ref, y0_ref, y1_ref, dega_ref, degb_ref, b_ref, out_ref):
    dinv = lax.rsqrt(dega_ref[:, 0:1] + degb_ref[:, 0:1] + 1.0)
    bv = b_ref[...]
    left = jnp.maximum((a0_ref[...] + y0_ref[...]) * dinv + bv[:, :H], 0.0)
    right = jnp.maximum((a1_ref[...] + y1_ref[...]) * dinv + bv[:, H:], 0.0)
    out_ref[...] = jnp.concatenate([left, right], axis=1)


def _combine(a0, a1, y0, y1, dega, degb, b2d):
    return pl.pallas_call(
        _comb_body,
        grid=(N // BM,),
        in_specs=[
            pl.BlockSpec((BM, H), lambda i: (i, 0)),
            pl.BlockSpec((BM, H), lambda i: (i, 0)),
            pl.BlockSpec((BM, H), lambda i: (i, 0)),
            pl.BlockSpec((BM, H), lambda i: (i, 0)),
            pl.BlockSpec((BM, H), lambda i: (i, 0)),
            pl.BlockSpec((BM, H), lambda i: (i, 0)),
            pl.BlockSpec((1, D), lambda i: (0, 0)),
        ],
        out_specs=pl.BlockSpec((BM, D), lambda i: (i, 0)),
        out_shape=jax.ShapeDtypeStruct((N, D), jnp.float32),
    )(a0, a1, y0, y1, dega, degb, b2d)


@functools.cache
def _degree_kernel():
    return pl.kernel(
        _degree_body,
        out_type=[jax.ShapeDtypeStruct((N, H), jnp.float32)] * 2,
        mesh=_mesh(),
        scratch_types=[
            pltpu.VMEM((CH,), jnp.int32),
            pltpu.VMEM((8,), jnp.int32),
            pltpu.VMEM((CH, H), jnp.float32),
            pltpu.VMEM_SHARED((N, H), jnp.float32),
        ],
    )


def _rowcopy(src, dst, s):
    rb = s * RB
    pltpu.sync_copy(src.at[pl.ds(rb, RB)], dst.at[pl.ds(rb, RB)])

    @pl.when(s == NT - 1)
    def _():
        pltpu.sync_copy(src.at[pl.ds(N - 16, 16)], dst.at[pl.ds(N - 16, 16)])


def _degree_body(dst_hbm, zeros_hbm, ones_hbm, dega_hbm, degb_hbm,
                 didx, didx_t, ones_v, deg_sh):
    c = lax.axis_index("c")
    s = lax.axis_index("s")

    def run(deg_hbm):
        _rowcopy(zeros_hbm, deg_sh, s)
        pltpu.sync_copy(ones_hbm, ones_v)
        plsc.subcore_barrier()
        base = c * (E // 2) + s * EPT2

        def body(i, carry):
            pltpu.sync_copy(dst_hbm.at[pl.ds(base + i * CH, CH)], didx)
            pltpu.sync_copy(ones_v, deg_sh.at[didx], add=True)
            return carry

        lax.fori_loop(0, NF2, body, 0)
        pltpu.sync_copy(dst_hbm.at[pl.ds(base + NF2 * CH, 8)], didx_t)
        pltpu.sync_copy(ones_v.at[pl.ds(0, 8)], deg_sh.at[didx_t], add=True)
        plsc.subcore_barrier()
        _rowcopy(deg_sh, deg_hbm, s)

    @pl.when(c == 0)
    def _():
        run(dega_hbm)

    @pl.when(c == 1)
    def _():
        run(degb_hbm)


@functools.cache
def _scatter_kernel():
    return pl.kernel(
        _scatter_body,
        out_type=[jax.ShapeDtypeStruct((N, H), jnp.float32)] * 2,
        mesh=_mesh(),
        scratch_types=[
            [pltpu.VMEM((CH,), jnp.int32)] * 3,
            [pltpu.VMEM((CH,), jnp.int32)] * 3,
            [pltpu.VMEM((CH, H), jnp.float32)] * 3,
            pltpu.VMEM((16,), jnp.int32),
            pltpu.VMEM((16,), jnp.int32),
            [pltpu.SemaphoreType.DMA] * 3,
            [pltpu.SemaphoreType.DMA] * 3,
            [pltpu.SemaphoreType.DMA] * 3,
            [pltpu.SemaphoreType.DMA] * 3,
            pltpu.VMEM_SHARED((N, H), jnp.float32),
        ],
    )


def _scatter_body(y0_hbm, y1_hbm, src_hbm, dst_hbm, zeros_hbm,
                  acc0_hbm, acc1_hbm,
                  sidx, didx, rows, sidx_t, didx_t,
                  gsem, ssem, issem, idsem, acc_sh):
    c = lax.axis_index("c")
    s = lax.axis_index("s")

    def run(y_hbm, acc_hbm):
        eb0 = s * EPT

        def chunk_step(k, cu, nx, pv):
            @pl.when(k >= 2)
            def _():
                pltpu.make_async_copy(rows[nx], acc_sh.at[didx[nx]], ssem[nx]).wait()

            @pl.when(k + 2 < NF)
            def _():
                pltpu.async_copy(src_hbm.at[pl.ds(eb0 + (k + 2) * CH, CH)],
                                 sidx[pv], issem[pv])

            @pl.when(k + 1 < NF)
            def _():
                pltpu.async_copy(dst_hbm.at[pl.ds(eb0 + (k + 1) * CH, CH)],
                                 didx[nx], idsem[nx])

            pltpu.make_async_copy(y_hbm.at[sidx[cu]], rows[cu], gsem[cu]).wait()

            @pl.when(k + 1 < NF)
            def _():
                pltpu.make_async_copy(src_hbm.at[pl.ds(eb0 + (k + 1) * CH, CH)],
                                      sidx[nx], issem[nx]).wait()
                pltpu.async_copy(y_hbm.at[sidx[nx]], rows[nx], gsem[nx])

            pltpu.make_async_copy(dst_hbm.at[pl.ds(eb0 + k * CH, CH)],
                                  didx[cu], idsem[cu]).wait()
            pltpu.async_copy(rows[cu], acc_sh.at[didx[cu]], ssem[cu], add=True)

        _rowcopy(zeros_hbm, acc_sh, s)
        plsc.subcore_barrier()

        pltpu.sync_copy(src_hbm.at[pl.ds(eb0, CH)], sidx[0])
        pltpu.async_copy(dst_hbm.at[pl.ds(eb0, CH)], didx[0], idsem[0])
        pltpu.async_copy(src_hbm.at[pl.ds(eb0 + CH, CH)], sidx[1], issem[1])
        pltpu.async_copy(y_hbm.at[sidx[0]], rows[0], gsem[0])

        def triple(j, carry):
            k = 3 * j
            chunk_step(k, 0, 1, 2)
            chunk_step(k + 1, 1, 2, 0)
            chunk_step(k + 2, 2, 0, 1)
            return carry

        lax.fori_loop(0, NF // 3, triple, 0)

        pltpu.make_async_copy(rows[(NF - 2) % 3], acc_sh.at[didx[(NF - 2) % 3]],
                              ssem[(NF - 2) % 3]).wait()
        pltpu.make_async_copy(rows[(NF - 1) % 3], acc_sh.at[didx[(NF - 1) % 3]],
                              ssem[(NF - 1) % 3]).wait()

        pltpu.sync_copy(src_hbm.at[pl.ds(eb0 + NF * CH, 16)], sidx_t)
        pltpu.sync_copy(dst_hbm.at[pl.ds(eb0 + NF * CH, 16)], didx_t)
        pltpu.async_copy(y_hbm.at[sidx_t], rows[0].at[pl.ds(0, 16)], gsem[0]).wait()
        pltpu.sync_copy(rows[0].at[pl.ds(0, 16)], acc_sh.at[didx_t], add=True)

        plsc.subcore_barrier()
        _rowcopy(acc_sh, acc_hbm, s)

    @pl.when(c == 0)
    def _():
        run(y0_hbm, acc0_hbm)

    @pl.when(c == 1)
    def _():
        run(y1_hbm, acc1_hbm)


def kernel(x, edge_index, W1, b1, W2, b2, W3, b3):
    src = edge_index[0]
    dst = edge_index[1]
    zerosH = jnp.zeros((N, H), jnp.float32)
    ones = jnp.ones((CH, H), jnp.float32)

    dega, degb = _degree_kernel()(dst, zerosH, ones)

    y0, y1 = _matmul_scale(x, W1, dega, degb)
    a0, a1 = _scatter_kernel()(y0, y1, src, dst, zerosH)
    for (w, b) in ((W2, b1), (W3, b2)):
        y0, y1 = _combine_matmul(a0, a1, y0, y1, dega, degb, b.reshape(1, D), w)
        a0, a1 = _scatter_kernel()(y0, y1, src, dst, zerosH)
    return _combine(a0, a1, y0, y1, dega, degb, b3.reshape(1, D))


# speedup vs baseline: 1.0534x; 1.0534x over previous
import functools

import jax
import jax.numpy as jnp
from jax import lax
from jax.experimental import pallas as pl
from jax.experimental.pallas import tpu as pltpu
from jax.experimental.pallas import tpu_sc as plsc

N = 10000
E = 160000
D = 256
H = 128
NT = 16
EPT = E // NT
RB = 624
CH = 128
NF = 78
NPAIR = NF // 2
EPT2 = E // (2 * NT)
NF2 = 39
BM = 1000

@functools.cache
def _mesh():
    return plsc.VectorSubcoreMesh(
        core_axis_name="c", subcore_axis_name="s", num_cores=2, num_subcores=NT)


def _mm_body(h_ref, w_ref, dega_ref, degb_ref, y0_ref, y1_ref):
    xw = jnp.dot(h_ref[...], w_ref[...], preferred_element_type=jnp.float32)
    dinv = lax.rsqrt(dega_ref[:, 0:1] + degb_ref[:, 0:1] + 1.0)
    y = xw * dinv
    y0_ref[...] = y[:, :H]
    y1_ref[...] = y[:, H:]


def _matmul_scale(h, w, dega, degb):
    return pl.pallas_call(
        _mm_body,
        grid=(N // BM,),
        in_specs=[
            pl.BlockSpec((BM, D), lambda i: (i, 0)),
            pl.BlockSpec((D, D), lambda i: (0, 0)),
            pl.BlockSpec((BM, H), lambda i: (i, 0)),
            pl.BlockSpec((BM, H), lambda i: (i, 0)),
        ],
        out_specs=[pl.BlockSpec((BM, H), lambda i: (i, 0))] * 2,
        out_shape=[jax.ShapeDtypeStruct((N, H), jnp.float32)] * 2,
    )(h, w, dega, degb)


def _cmm_body(a0_ref, a1_ref, y0_ref, y1_ref, dega_ref, degb_ref, b_ref, w_ref,
              y0o_ref, y1o_ref):
    dinv = lax.rsqrt(dega_ref[:, 0:1] + degb_ref[:, 0:1] + 1.0)
    bv = b_ref[...]
    left = jnp.maximum((a0_ref[...] + y0_ref[...]) * dinv + bv[:, :H], 0.0)
    right = jnp.maximum((a1_ref[...] + y1_ref[...]) * dinv + bv[:, H:], 0.0)
    h = jnp.concatenate([left, right], axis=1)
    y = jnp.dot(h, w_ref[...], preferred_element_type=jnp.float32) * dinv
    y0o_ref[...] = y[:, :H]
    y1o_ref[...] = y[:, H:]


def _combine_matmul(a0, a1, y0, y1, dega, degb, b2d, w):
    blk = pl.BlockSpec((BM, H), lambda i: (i, 0))
    return pl.pallas_call(
        _cmm_body,
        grid=(N // BM,),
        in_specs=[
            blk, blk, blk, blk, blk, blk,
            pl.BlockSpec((1, D), lambda i: (0, 0)),
            pl.BlockSpec((D, D), lambda i: (0, 0)),
        ],
        out_specs=[blk, blk],
        out_shape=[jax.ShapeDtypeStruct((N, H), jnp.float32)] * 2,
    )(a0, a1, y0, y1, dega, degb, b2d, w)


def _comb_body(a0_ref, a1_ref, y0_ref, y1_ref, dega_ref, degb_ref, b_ref, out_ref):
    dinv = lax.rsqrt(dega_ref[:, 0:1] + degb_ref[:, 0:1] + 1.0)
    bv = b_ref[...]
    left = jnp.maximum((a0_ref[...] + y0_ref[...]) * dinv + bv[:, :H], 0.0)
    right = jnp.maximum((a1_ref[...] + y1_ref[...]) * dinv + bv[:, H:], 0.0)
    out_ref[...] = jnp.concatenate([left, right], axis=1)


def _combine(a0, a1, y0, y1, dega, degb, b2d):
    return pl.pallas_call(
        _comb_body,
        grid=(N // BM,),
        in_specs=[
            pl.BlockSpec((BM, H), lambda i: (i, 0)),
            pl.BlockSpec((BM, H), lambda i: (i, 0)),
            pl.BlockSpec((BM, H), lambda i: (i, 0)),
            pl.BlockSpec((BM, H), lambda i: (i, 0)),
            pl.BlockSpec((BM, H), lambda i: (i, 0)),
            pl.BlockSpec((BM, H), lambda i: (i, 0)),
            pl.BlockSpec((1, D), lambda i: (0, 0)),
        ],
        out_specs=pl.BlockSpec((BM, D), lambda i: (i, 0)),
        out_shape=jax.ShapeDtypeStruct((N, D), jnp.float32),
    )(a0, a1, y0, y1, dega, degb, b2d)


@functools.cache
def _degree_kernel():
    return pl.kernel(
        _degree_body,
        out_type=[jax.ShapeDtypeStruct((N, H), jnp.float32)] * 2,
        mesh=_mesh(),
        scratch_types=[
            pltpu.VMEM((CH,), jnp.int32),
            pltpu.VMEM((CH,), jnp.int32),
            pltpu.VMEM((8,), jnp.int32),
            pltpu.VMEM((CH, H), jnp.float32),
            pltpu.SemaphoreType.DMA,
            pltpu.SemaphoreType.DMA,
            pltpu.VMEM_SHARED((N, H), jnp.float32),
        ],
    )


def _rowcopy(src, dst, s):
    rb = s * RB
    pltpu.sync_copy(src.at[pl.ds(rb, RB)], dst.at[pl.ds(rb, RB)])

    @pl.when(s == NT - 1)
    def _():
        pltpu.sync_copy(src.at[pl.ds(N - 16, 16)], dst.at[pl.ds(N - 16, 16)])


def _degree_body(dst_hbm, zeros_hbm, ones_hbm, dega_hbm, degb_hbm,
                 didx_a, didx_b, didx_t, ones_v, isem_a, isem_b, deg_sh):
    c = lax.axis_index("c")
    s = lax.axis_index("s")

    def run(deg_hbm):
        _rowcopy(zeros_hbm, deg_sh, s)
        pltpu.sync_copy(ones_hbm, ones_v)
        plsc.subcore_barrier()
        base = c * (E // 2) + s * EPT2

        def start_didx(i, didx, isem):
            pltpu.async_copy(dst_hbm.at[pl.ds(base + i * CH, CH)], didx, isem)

        def drain_didx(i, didx, isem):
            pltpu.make_async_copy(dst_hbm.at[pl.ds(base + i * CH, CH)], didx, isem).wait()

        pltpu.sync_copy(dst_hbm.at[pl.ds(base, CH)], didx_a)

        def pair(j, carry):
            i0 = 2 * j
            start_didx(i0 + 1, didx_b, isem_b)
            pltpu.sync_copy(ones_v, deg_sh.at[didx_a], add=True)
            drain_didx(i0 + 1, didx_b, isem_b)
            start_didx(i0 + 2, didx_a, isem_a)
            pltpu.sync_copy(ones_v, deg_sh.at[didx_b], add=True)
            drain_didx(i0 + 2, didx_a, isem_a)
            return carry

        lax.fori_loop(0, (NF2 - 1) // 2, pair, 0)
        pltpu.sync_copy(ones_v, deg_sh.at[didx_a], add=True)
        pltpu.sync_copy(dst_hbm.at[pl.ds(base + NF2 * CH, 8)], didx_t)
        pltpu.sync_copy(ones_v.at[pl.ds(0, 8)], deg_sh.at[didx_t], add=True)
        plsc.subcore_barrier()
        _rowcopy(deg_sh, deg_hbm, s)

    @pl.when(c == 0)
    def _():
        run(dega_hbm)

    @pl.when(c == 1)
    def _():
        run(degb_hbm)


@functools.cache
def _scatter_kernel():
    return pl.kernel(
        _scatter_body,
        out_type=[jax.ShapeDtypeStruct((N, H), jnp.float32)] * 2,
        mesh=_mesh(),
        scratch_types=[
            pltpu.VMEM((CH,), jnp.int32),
            pltpu.VMEM((CH,), jnp.int32),
            pltpu.VMEM((CH,), jnp.int32),
            pltpu.VMEM((CH,), jnp.int32),
            pltpu.VMEM((16,), jnp.int32),
            pltpu.VMEM((16,), jnp.int32),
            pltpu.VMEM((CH, H), jnp.float32),
            pltpu.VMEM((CH, H), jnp.float32),
            pltpu.SemaphoreType.DMA,
            pltpu.SemaphoreType.DMA,
            pltpu.SemaphoreType.DMA,
            pltpu.SemaphoreType.DMA,
            pltpu.VMEM_SHARED((N, H), jnp.float32),
        ],
    )


def _scatter_body(y0_hbm, y1_hbm, src_hbm, dst_hbm, zeros_hbm,
                  acc0_hbm, acc1_hbm,
                  sidx_a, didx_a, sidx_b, didx_b, sidx_t, didx_t,
                  rows_a, rows_b,
                  gsem_a, gsem_b, isem_a, isem_b, acc_sh):
    c = lax.axis_index("c")
    s = lax.axis_index("s")

    def run(y_hbm, acc_hbm):
        eb0 = s * EPT

        def start_idx(i, sidx, didx, isem):
            pltpu.async_copy(src_hbm.at[pl.ds(eb0 + i * CH, CH)], sidx, isem)
            pltpu.async_copy(dst_hbm.at[pl.ds(eb0 + i * CH, CH)], didx, isem)

        def drain_idx(i, sidx, didx, isem):
            pltpu.make_async_copy(src_hbm.at[pl.ds(eb0 + i * CH, CH)], sidx, isem).wait()
            pltpu.make_async_copy(dst_hbm.at[pl.ds(eb0 + i * CH, CH)], didx, isem).wait()

        _rowcopy(zeros_hbm, acc_sh, s)
        plsc.subcore_barrier()

        pltpu.sync_copy(src_hbm.at[pl.ds(eb0, CH)], sidx_a)
        pltpu.sync_copy(dst_hbm.at[pl.ds(eb0, CH)], didx_a)
        pltpu.async_copy(y_hbm.at[sidx_a], rows_a, gsem_a)

        def pair(j, carry):
            i0 = 2 * j
            start_idx(i0 + 1, sidx_b, didx_b, isem_b)
            pltpu.make_async_copy(y_hbm.at[sidx_a], rows_a, gsem_a).wait()
            drain_idx(i0 + 1, sidx_b, didx_b, isem_b)
            pltpu.async_copy(y_hbm.at[sidx_b], rows_b, gsem_b)
            pltpu.sync_copy(rows_a, acc_sh.at[didx_a], add=True)

            @pl.when(j < NPAIR - 1)
            def _():
                start_idx(i0 + 2, sidx_a, didx_a, isem_a)
                drain_idx(i0 + 2, sidx_a, didx_a, isem_a)
                pltpu.async_copy(y_hbm.at[sidx_a], rows_a, gsem_a)

            pltpu.make_async_copy(y_hbm.at[sidx_b], rows_b, gsem_b).wait()
            pltpu.sync_copy(rows_b, acc_sh.at[didx_b], add=True)
            return carry

        lax.fori_loop(0, NPAIR, pair, 0)

        pltpu.sync_copy(src_hbm.at[pl.ds(eb0 + NF * CH, 16)], sidx_t)
        pltpu.sync_copy(dst_hbm.at[pl.ds(eb0 + NF * CH, 16)], didx_t)
        pltpu.async_copy(y_hbm.at[sidx_t], rows_a.at[pl.ds(0, 16)], gsem_a).wait()
        pltpu.sync_copy(rows_a.at[pl.ds(0, 16)], acc_sh.at[didx_t], add=True)

        plsc.subcore_barrier()
        _rowcopy(acc_sh, acc_hbm, s)

    @pl.when(c == 0)
    def _():
        run(y0_hbm, acc0_hbm)

    @pl.when(c == 1)
    def _():
        run(y1_hbm, acc1_hbm)


def kernel(x, edge_index, W1, b1, W2, b2, W3, b3):
    src = edge_index[0]
    dst = edge_index[1]
    zerosH = jnp.zeros((N, H), jnp.float32)
    ones = jnp.ones((CH, H), jnp.float32)

    dega, degb = _degree_kernel()(dst, zerosH, ones)

    y0, y1 = _matmul_scale(x, W1, dega, degb)
    a0, a1 = _scatter_kernel()(y0, y1, src, dst, zerosH)
    for (w, b) in ((W2, b1), (W3, b2)):
        y0, y1 = _combine_matmul(a0, a1, y0, y1, dega, degb, b.reshape(1, D), w)
        a0, a1 = _scatter_kernel()(y0, y1, src, dst, zerosH)
    return _combine(a0, a1, y0, y1, dega, degb, b3.reshape(1, D))


# speedup vs baseline: 1.0593x; 1.0056x over previous
import functools

import jax
import jax.numpy as jnp
from jax import lax
from jax.experimental import pallas as pl
from jax.experimental.pallas import tpu as pltpu
from jax.experimental.pallas import tpu_sc as plsc

N = 10000
E = 160000
D = 256
H = 128
NT = 16
EPT = E // NT
RB = 624
CH = 128
NF = 78
NPAIR = NF // 2
EPT2 = E // (2 * NT)
NF2 = 39
BM = 1000

@functools.cache
def _mesh():
    return plsc.VectorSubcoreMesh(
        core_axis_name="c", subcore_axis_name="s", num_cores=2, num_subcores=NT)


def _mm_body(h_ref, w_ref, dega_ref, degb_ref, y0_ref, y1_ref):
    xw = jnp.dot(h_ref[...], w_ref[...], preferred_element_type=jnp.float32)
    dinv = lax.rsqrt(dega_ref[:, 0:1] + degb_ref[:, 0:1] + 1.0)
    y = xw * dinv
    y0_ref[...] = y[:, :H]
    y1_ref[...] = y[:, H:]


def _matmul_scale(h, w, dega, degb):
    return pl.pallas_call(
        _mm_body,
        grid=(N // BM,),
        in_specs=[
            pl.BlockSpec((BM, D), lambda i: (i, 0)),
            pl.BlockSpec((D, D), lambda i: (0, 0)),
            pl.BlockSpec((BM, H), lambda i: (i, 0)),
            pl.BlockSpec((BM, H), lambda i: (i, 0)),
        ],
        out_specs=[pl.BlockSpec((BM, H), lambda i: (i, 0))] * 2,
        out_shape=[jax.ShapeDtypeStruct((N, H), jnp.float32)] * 2,
    )(h, w, dega, degb)


def _cmm_body(a0_ref, a1_ref, y0_ref, y1_ref, dega_ref, degb_ref, b_ref, w_ref,
              y0o_ref, y1o_ref):
    dinv = lax.rsqrt(dega_ref[:, 0:1] + degb_ref[:, 0:1] + 1.0)
    bv = b_ref[...]
    left = jnp.maximum((a0_ref[...] + y0_ref[...]) * dinv + bv[:, :H], 0.0)
    right = jnp.maximum((a1_ref[...] + y1_ref[...]) * dinv + bv[:, H:], 0.0)
    h = jnp.concatenate([left, right], axis=1)
    y = jnp.dot(h, w_ref[...], preferred_element_type=jnp.float32) * dinv
    y0o_ref[...] = y[:, :H]
    y1o_ref[...] = y[:, H:]


def _combine_matmul(a0, a1, y0, y1, dega, degb, b2d, w):
    blk = pl.BlockSpec((BM, H), lambda i: (i, 0))
    return pl.pallas_call(
        _cmm_body,
        grid=(N // BM,),
        in_specs=[
            blk, blk, blk, blk, blk, blk,
            pl.BlockSpec((1, D), lambda i: (0, 0)),
            pl.BlockSpec((D, D), lambda i: (0, 0)),
        ],
        out_specs=[blk, blk],
        out_shape=[jax.ShapeDtypeStruct((N, H), jnp.float32)] * 2,
    )(a0, a1, y0, y1, dega, degb, b2d, w)


def _comb_body(a0_ref, a1_ref, y0_ref, y1_ref, dega_ref, degb_ref, b_ref, out_ref):
    dinv = lax.rsqrt(dega_ref[:, 0:1] + degb_ref[:, 0:1] + 1.0)
    bv = b_ref[...]
    left = jnp.maximum((a0_ref[...] + y0_ref[...]) * dinv + bv[:, :H], 0.0)
    right = jnp.maximum((a1_ref[...] + y1_ref[...]) * dinv + bv[:, H:], 0.0)
    out_ref[...] = jnp.concatenate([left, right], axis=1)


def _combine(a0, a1, y0, y1, dega, degb, b2d):
    return pl.pallas_call(
        _comb_body,
        grid=(N // BM,),
        in_specs=[
            pl.BlockSpec((BM, H), lambda i: (i, 0)),
            pl.BlockSpec((BM, H), lambda i: (i, 0)),
            pl.BlockSpec((BM, H), lambda i: (i, 0)),
            pl.BlockSpec((BM, H), lambda i: (i, 0)),
            pl.BlockSpec((BM, H), lambda i: (i, 0)),
            pl.BlockSpec((BM, H), lambda i: (i, 0)),
            pl.BlockSpec((1, D), lambda i: (0, 0)),
        ],
        out_specs=pl.BlockSpec((BM, D), lambda i: (i, 0)),
        out_shape=jax.ShapeDtypeStruct((N, D), jnp.float32),
    )(a0, a1, y0, y1, dega, degb, b2d)


@functools.cache
def _degree_kernel():
    return pl.kernel(
        _degree_body,
        out_type=[jax.ShapeDtypeStruct((N, H), jnp.float32)] * 2,
        mesh=_mesh(),
        scratch_types=[
            pltpu.VMEM((CH,), jnp.int32),
            pltpu.VMEM((CH,), jnp.int32),
            pltpu.VMEM((8,), jnp.int32),
            pltpu.VMEM((CH, H), jnp.float32),
            pltpu.SemaphoreType.DMA,
            pltpu.SemaphoreType.DMA,
            pltpu.VMEM_SHARED((N, H), jnp.float32),
        ],
    )


def _rowcopy(src, dst, s):
    rb = s * RB
    pltpu.sync_copy(src.at[pl.ds(rb, RB)], dst.at[pl.ds(rb, RB)])

    @pl.when(s == NT - 1)
    def _():
        pltpu.sync_copy(src.at[pl.ds(N - 16, 16)], dst.at[pl.ds(N - 16, 16)])


def _degree_body(dst_hbm, zeros_hbm, ones_hbm, dega_hbm, degb_hbm,
                 didx_a, didx_b, didx_t, ones_v, isem_a, isem_b, deg_sh):
    c = lax.axis_index("c")
    s = lax.axis_index("s")

    def run(deg_hbm):
        base = c * (E // 2) + s * EPT2

        def start_didx(i, didx, isem):
            pltpu.async_copy(dst_hbm.at[pl.ds(base + i * CH, CH)], didx, isem)

        def drain_didx(i, didx, isem):
            pltpu.make_async_copy(dst_hbm.at[pl.ds(base + i * CH, CH)], didx, isem).wait()

        start_didx(0, didx_a, isem_a)
        pltpu.async_copy(ones_hbm, ones_v, isem_b)
        _rowcopy(zeros_hbm, deg_sh, s)
        drain_didx(0, didx_a, isem_a)
        pltpu.make_async_copy(ones_hbm, ones_v, isem_b).wait()
        plsc.subcore_barrier()

        def pair(j, carry):
            i0 = 2 * j
            start_didx(i0 + 1, didx_b, isem_b)
            pltpu.sync_copy(ones_v, deg_sh.at[didx_a], add=True)
            drain_didx(i0 + 1, didx_b, isem_b)
            start_didx(i0 + 2, didx_a, isem_a)
            pltpu.sync_copy(ones_v, deg_sh.at[didx_b], add=True)
            drain_didx(i0 + 2, didx_a, isem_a)
            return carry

        lax.fori_loop(0, (NF2 - 1) // 2, pair, 0)
        pltpu.sync_copy(ones_v, deg_sh.at[didx_a], add=True)
        pltpu.sync_copy(dst_hbm.at[pl.ds(base + NF2 * CH, 8)], didx_t)
        pltpu.sync_copy(ones_v.at[pl.ds(0, 8)], deg_sh.at[didx_t], add=True)
        plsc.subcore_barrier()
        _rowcopy(deg_sh, deg_hbm, s)

    @pl.when(c == 0)
    def _():
        run(dega_hbm)

    @pl.when(c == 1)
    def _():
        run(degb_hbm)


@functools.cache
def _scatter_kernel():
    return pl.kernel(
        _scatter_body,
        out_type=[jax.ShapeDtypeStruct((N, H), jnp.float32)] * 2,
        mesh=_mesh(),
        scratch_types=[
            pltpu.VMEM((CH,), jnp.int32),
            pltpu.VMEM((CH,), jnp.int32),
            pltpu.VMEM((CH,), jnp.int32),
            pltpu.VMEM((CH,), jnp.int32),
            pltpu.VMEM((16,), jnp.int32),
            pltpu.VMEM((16,), jnp.int32),
            pltpu.VMEM((CH, H), jnp.float32),
            pltpu.VMEM((CH, H), jnp.float32),
            pltpu.SemaphoreType.DMA,
            pltpu.SemaphoreType.DMA,
            pltpu.SemaphoreType.DMA,
            pltpu.SemaphoreType.DMA,
            pltpu.VMEM_SHARED((N, H), jnp.float32),
        ],
    )


def _scatter_body(y0_hbm, y1_hbm, src_hbm, dst_hbm, zeros_hbm,
                  acc0_hbm, acc1_hbm,
                  sidx_a, didx_a, sidx_b, didx_b, sidx_t, didx_t,
                  rows_a, rows_b,
                  gsem_a, gsem_b, isem_a, isem_b, acc_sh):
    c = lax.axis_index("c")
    s = lax.axis_index("s")

    def run(y_hbm, acc_hbm):
        eb0 = s * EPT

        def start_idx(i, sidx, didx, isem):
            pltpu.async_copy(src_hbm.at[pl.ds(eb0 + i * CH, CH)], sidx, isem)
            pltpu.async_copy(dst_hbm.at[pl.ds(eb0 + i * CH, CH)], didx, isem)

        def drain_idx(i, sidx, didx, isem):
            pltpu.make_async_copy(src_hbm.at[pl.ds(eb0 + i * CH, CH)], sidx, isem).wait()
            pltpu.make_async_copy(dst_hbm.at[pl.ds(eb0 + i * CH, CH)], didx, isem).wait()

        pltpu.sync_copy(src_hbm.at[pl.ds(eb0, CH)], sidx_a)
        pltpu.async_copy(dst_hbm.at[pl.ds(eb0, CH)], didx_a, isem_a)
        pltpu.async_copy(y_hbm.at[sidx_a], rows_a, gsem_a)

        _rowcopy(zeros_hbm, acc_sh, s)
        pltpu.make_async_copy(dst_hbm.at[pl.ds(eb0, CH)], didx_a, isem_a).wait()
        plsc.subcore_barrier()

        def pair(j, carry):
            i0 = 2 * j
            start_idx(i0 + 1, sidx_b, didx_b, isem_b)
            pltpu.make_async_copy(y_hbm.at[sidx_a], rows_a, gsem_a).wait()
            drain_idx(i0 + 1, sidx_b, didx_b, isem_b)
            pltpu.async_copy(y_hbm.at[sidx_b], rows_b, gsem_b)
            pltpu.sync_copy(rows_a, acc_sh.at[didx_a], add=True)

            @pl.when(j < NPAIR - 1)
            def _():
                start_idx(i0 + 2, sidx_a, didx_a, isem_a)
                drain_idx(i0 + 2, sidx_a, didx_a, isem_a)
                pltpu.async_copy(y_hbm.at[sidx_a], rows_a, gsem_a)

            pltpu.make_async_copy(y_hbm.at[sidx_b], rows_b, gsem_b).wait()
            pltpu.sync_copy(rows_b, acc_sh.at[didx_b], add=True)
            return carry

        lax.fori_loop(0, NPAIR, pair, 0)

        pltpu.sync_copy(src_hbm.at[pl.ds(eb0 + NF * CH, 16)], sidx_t)
        pltpu.sync_copy(dst_hbm.at[pl.ds(eb0 + NF * CH, 16)], didx_t)
        pltpu.async_copy(y_hbm.at[sidx_t], rows_a.at[pl.ds(0, 16)], gsem_a).wait()
        pltpu.sync_copy(rows_a.at[pl.ds(0, 16)], acc_sh.at[didx_t], add=True)

        plsc.subcore_barrier()
        _rowcopy(acc_sh, acc_hbm, s)

    @pl.when(c == 0)
    def _():
        run(y0_hbm, acc0_hbm)

    @pl.when(c == 1)
    def _():
        run(y1_hbm, acc1_hbm)


def kernel(x, edge_index, W1, b1, W2, b2, W3, b3):
    src = edge_index[0]
    dst = edge_index[1]
    zerosH = jnp.zeros((N, H), jnp.float32)
    ones = jnp.ones((CH, H), jnp.float32)

    dega, degb = _degree_kernel()(dst, zerosH, ones)

    y0, y1 = _matmul_scale(x, W1, dega, degb)
    a0, a1 = _scatter_kernel()(y0, y1, src, dst, zerosH)
    for (w, b) in ((W2, b1), (W3, b2)):
        y0, y1 = _combine_matmul(a0, a1, y0, y1, dega, degb, b.reshape(1, D), w)
        a0, a1 = _scatter_kernel()(y0, y1, src, dst, zerosH)
    return _combine(a0, a1, y0, y1, dega, degb, b3.reshape(1, D))
